# baseline (speedup 1.0000x reference)
import functools

import jax
import jax.numpy as jnp
from jax import lax
from jax.experimental import pallas as pl
from jax.experimental.pallas import tpu as pltpu
from jax.experimental.pallas import tpu_sc as plsc

_SCALE = 10.0
_NC = 2
_NS = 16
_NW = _NC * _NS
_CH = 128
_NBUF = 4


@functools.lru_cache(maxsize=None)
def _make_gather_scale(V, D, S0, S1):
  B = S0 * S1
  n_st = S0 // _CH
  n_dt = D // 8
  assert B % (_NW * _CH) == 0 and S0 % _CH == 0 and D % 8 == 0
  n_ch = B // (_NW * _CH)
  n_tiles = B // _CH * n_dt
  assert n_ch % _NBUF == 0 and n_ch >= 2 * _NBUF
  mesh = plsc.VectorSubcoreMesh(core_axis_name="c", subcore_axis_name="s")

  @functools.partial(
      pl.kernel,
      mesh=mesh,
      out_type=jax.ShapeDtypeStruct((n_tiles, 8, _CH), jnp.float32),
      scratch_types=[
          pltpu.VMEM((n_ch, _CH), jnp.int32),
          [pltpu.VMEM((_CH, D), jnp.float32)] * _NBUF,
          [pltpu.VMEM((n_dt, 8, _CH), jnp.float32)] * _NBUF,
          [pltpu.SemaphoreType.DMA] * _NBUF,
          [pltpu.SemaphoreType.DMA] * _NBUF,
      ],
      compiler_params=pltpu.CompilerParams(
          use_tc_tiling_on_sc=False, needs_layout_passes=False),
  )
  def gather_scale(table_hbm, idx_hbm, out_hbm, idx_v, in_b, out_b,
                   in_sems, out_sems):
    wid = lax.axis_index("s") * _NC + lax.axis_index("c")
    ch_base = wid * n_ch
    pltpu.sync_copy(idx_hbm.at[pl.ds(ch_base, n_ch)], idx_v)

    lanes = lax.iota(jnp.int32, 16)

    def gather(cl, bi):
      return pltpu.async_copy(
          table_hbm.at[idx_v.at[cl]], in_b[bi], in_sems[bi])

    def out_tile_base(cl):
      c = ch_base + cl
      s1 = lax.div(c, n_st)
      st = lax.rem(c, n_st)
      return (s1 * n_dt) * n_st + st

    def writeback(cl, bi):
      base = out_tile_base(cl)
      for dt in range(n_dt):
        pltpu.async_copy(
            out_b[bi].at[pl.ds(dt, 1)],
            out_hbm.at[pl.ds(base + dt * n_st, 1)],
            out_sems[bi])

    def wait_writeback(cl, bi):
      base = out_tile_base(cl)
      for dt in range(n_dt):
        pltpu.make_async_copy(
            out_b[bi].at[pl.ds(dt, 1)],
            out_hbm.at[pl.ds(base + dt * n_st, 1)],
            out_sems[bi]).wait()

    for bi in range(_NBUF):
      gather(bi, bi)

    def outer(c0, carry):
      for bi in range(_NBUF):
        cl = c0 + bi
        pltpu.make_async_copy(
            table_hbm.at[idx_v.at[cl]], in_b[bi], in_sems[bi]).wait()
        @pl.when(cl >= _NBUF)
        def _():
          wait_writeback(cl - _NBUF, bi)

        def group_body(g, carry2):
          rows = g * 16 + lanes
          for d in range(D):
            v = plsc.load_gather(
                in_b[bi], [rows, jnp.full((16,), d, jnp.int32)])
            out_b[bi][d // 8, d % 8, pl.ds(g * 16, 16)] = v * _SCALE
          return carry2

        lax.fori_loop(0, _CH // 16, group_body, 0)

        writeback(cl, bi)

        @pl.when(cl + _NBUF < n_ch)
        def _():
          gather(cl + _NBUF, bi)
      return carry

    lax.fori_loop(0, n_ch // _NBUF, lambda i, cr: outer(i * _NBUF, cr), 0)

    for bi in range(_NBUF):
      wait_writeback(n_ch - _NBUF + bi, bi)

  return gather_scale


def kernel(x, weight):
  S0, S1 = x.shape
  V, D = weight.shape
  B = S0 * S1
  idx = x.T.reshape(B // _CH, _CH).astype(jnp.int32)
  out_t = _make_gather_scale(V, D, S0, S1)(weight, idx)
  n_st = S0 // _CH
  n_dt = D // 8
  out = out_t.reshape(S1, n_dt, n_st, 8, _CH).transpose(2, 4, 0, 1, 3)
  return out.reshape(S0, S1, D)


# speedup vs baseline: 1.1393x; 1.1393x over previous
import functools

import jax
import jax.numpy as jnp
from jax import lax
from jax.experimental import pallas as pl
from jax.experimental.pallas import tpu as pltpu
from jax.experimental.pallas import tpu_sc as plsc

_SCALE = 10.0
_NC = 2
_NS = 16
_NW = _NC * _NS
_CH = 128
_NBUF = 4


@functools.lru_cache(maxsize=None)
def _make_gather_scale(V, D, S0, S1):
  B = S0 * S1
  n_st = S0 // _CH
  n_dt = D // 8
  assert B % (_NW * _CH) == 0 and S0 % _CH == 0 and D % 8 == 0
  n_ch = B // (_NW * _CH)
  n_tiles = B // _CH * n_dt
  assert n_ch % _NBUF == 0 and n_ch >= 2 * _NBUF
  mesh = plsc.VectorSubcoreMesh(core_axis_name="c", subcore_axis_name="s")

  @functools.partial(
      pl.kernel,
      mesh=mesh,
      out_type=jax.ShapeDtypeStruct((n_tiles, 8, _CH), jnp.float32),
      scratch_types=[
          pltpu.VMEM((n_ch, _CH), jnp.int32),
          [pltpu.VMEM((_CH, D), jnp.float32)] * _NBUF,
          [pltpu.VMEM((n_dt, 8, _CH), jnp.float32)] * _NBUF,
          [pltpu.SemaphoreType.DMA] * _NBUF,
          [pltpu.SemaphoreType.DMA] * _NBUF,
      ],
      compiler_params=pltpu.CompilerParams(
          use_tc_tiling_on_sc=False, needs_layout_passes=False),
  )
  def gather_scale(table_hbm, idx_hbm, out_hbm, idx_v, in_b, out_b,
                   in_sems, out_sems):
    wid = lax.axis_index("s") * _NC + lax.axis_index("c")
    ch_base = wid * n_ch
    pltpu.sync_copy(idx_hbm.at[pl.ds(ch_base, n_ch)], idx_v)

    lanes = lax.iota(jnp.int32, 16)

    def gather(cl, bi):
      return pltpu.async_copy(
          table_hbm.at[idx_v.at[cl]], in_b[bi], in_sems[bi])

    def out_tile_base(cl):
      c = ch_base + cl
      s1 = lax.div(c, n_st)
      st = lax.rem(c, n_st)
      return (s1 * n_dt) * n_st + st

    def writeback(cl, bi):
      base = out_tile_base(cl)
      for dt in range(n_dt):
        pltpu.async_copy(
            out_b[bi].at[pl.ds(dt, 1)],
            out_hbm.at[pl.ds(base + dt * n_st, 1)],
            out_sems[bi])

    def wait_writeback(cl, bi):
      base = out_tile_base(cl)
      for dt in range(n_dt):
        pltpu.make_async_copy(
            out_b[bi].at[pl.ds(dt, 1)],
            out_hbm.at[pl.ds(base + dt * n_st, 1)],
            out_sems[bi]).wait()

    for bi in range(_NBUF):
      gather(bi, bi)

    def outer(c0, carry):
      for bi in range(_NBUF):
        cl = c0 + bi
        pltpu.make_async_copy(
            table_hbm.at[idx_v.at[cl]], in_b[bi], in_sems[bi]).wait()
        @pl.when(cl >= _NBUF)
        def _():
          wait_writeback(cl - _NBUF, bi)

        @plsc.parallel_loop(0, _CH // 16, unroll=2)
        def _(g):
          rows = g * 16 + lanes
          for d in range(D):
            v = plsc.load_gather(
                in_b[bi], [rows, jnp.full((16,), d, jnp.int32)])
            out_b[bi][d // 8, d % 8, pl.ds(g * 16, 16)] = v * _SCALE

        writeback(cl, bi)

        @pl.when(cl + _NBUF < n_ch)
        def _():
          gather(cl + _NBUF, bi)
      return carry

    lax.fori_loop(0, n_ch // _NBUF, lambda i, cr: outer(i * _NBUF, cr), 0)

    for bi in range(_NBUF):
      wait_writeback(n_ch - _NBUF + bi, bi)

  return gather_scale


def kernel(x, weight):
  S0, S1 = x.shape
  V, D = weight.shape
  B = S0 * S1
  idx = x.T.reshape(B // _CH, _CH).astype(jnp.int32)
  out_t = _make_gather_scale(V, D, S0, S1)(weight, idx)
  n_st = S0 // _CH
  n_dt = D // 8
  out = out_t.reshape(S1, n_dt, n_st, 8, _CH).transpose(2, 4, 0, 1, 3)
  return out.reshape(S0, S1, D)
